# Initial kernel scaffold; baseline (speedup 1.0000x reference)
import jax
import jax.numpy as jnp
from jax.experimental import pallas as pl


def kernel(x_new, rotary_cos, rotary_sin, past_k, past_v, Wq, bq, Wk, bk, Wv, bv, Wo, bo, past_lengths, new_token_counts, valid_new_mask):
    raise NotImplementedError("write your pallas kernel here")


# speedup vs baseline: 2.9198x; 2.9198x over previous
import jax
import jax.numpy as jnp
from jax import lax
from jax.experimental import pallas as pl
from jax.experimental.pallas import tpu as pltpu

B, TN, D = 16, 16, 1024
H, DH = 16, 64
LP = 2048
NEG = -1e30
_DN = (((1,), (1,)), ((), ()))


def _proj_kernel(x_ref, wq_ref, wk_ref, wv_ref, bq_ref, bk_ref, bv_ref,
                 cos_ref, sin_ref, q_ref, k_ref, v_ref):
    x = x_ref[...]
    cos = cos_ref[...]
    sin = sin_ref[...]

    def rope(y):
        parts = []
        for h in range(H):
            yh = y[:, h * DH:(h + 1) * DH]
            rh = jnp.concatenate([yh[:, DH // 2:], yh[:, :DH // 2]], axis=-1)
            parts.append(yh * cos + rh * sin)
        return jnp.concatenate(parts, axis=-1)

    q = lax.dot_general(x, wq_ref[...], _DN, preferred_element_type=jnp.float32) + bq_ref[...]
    q_ref[...] = rope(q)
    k = lax.dot_general(x, wk_ref[...], _DN, preferred_element_type=jnp.float32) + bk_ref[...]
    k_ref[...] = rope(k)
    v_ref[...] = lax.dot_general(x, wv_ref[...], _DN, preferred_element_type=jnp.float32) + bv_ref[...]


def _attn_kernel(lens_ref, q_ref, kn_ref, vn_ref, pk_ref, pv_ref,
                 wo_ref, bo_ref, o_ref):
    b = pl.program_id(0)
    plen = lens_ref[0, b]
    nc = lens_ref[1, b]
    q = q_ref[0]

    CW = LP // 2 + TN // 2
    r2 = 2 * lax.broadcasted_iota(jnp.int32, (1, LP // 2), 1)
    ebias = jnp.where(r2 < plen, 0.0, NEG)
    obias = jnp.where(r2 + 1 < plen, 0.0, NEG)
    t2i = 2 * lax.broadcasted_iota(jnp.int32, (1, TN // 2), 1)
    enb = jnp.where(t2i < nc, 0.0, NEG)
    onb = jnp.where(t2i + 1 < nc, 0.0, NEG)
    btop = jnp.concatenate([ebias, enb], axis=1)
    bbot = jnp.concatenate([obias, onb], axis=1)
    bias2 = jnp.concatenate([jnp.broadcast_to(btop, (TN, CW)),
                             jnp.broadcast_to(bbot, (TN, CW))], axis=0)
    scale = 1.0 / 8.0
    zpad = jnp.zeros((TN, DH), jnp.float32)

    outs = []
    for h in range(H):
        sl = slice(h * DH, (h + 1) * DH)
        qh = q[:, sl] * scale
        kf = jnp.concatenate([pk_ref[0, h], kn_ref[0, h]], axis=0)
        vf = jnp.concatenate([pv_ref[0, h], vn_ref[0, h]], axis=0)
        qblk = jnp.concatenate(
            [jnp.concatenate([qh, zpad], axis=1),
             jnp.concatenate([zpad, qh], axis=1)], axis=0)
        s2 = lax.dot_general(qblk, kf, _DN,
                             preferred_element_type=jnp.float32) + bias2
        m = jnp.maximum(jnp.max(s2[:TN], axis=-1, keepdims=True),
                        jnp.max(s2[TN:], axis=-1, keepdims=True))
        m2 = jnp.concatenate([m, m], axis=0)
        p2 = jnp.exp(s2 - m2)
        denom = (jnp.sum(p2[:TN], axis=-1, keepdims=True)
                 + jnp.sum(p2[TN:], axis=-1, keepdims=True))
        t2 = jnp.dot(p2, vf, preferred_element_type=jnp.float32)
        oh = (t2[:TN, :DH] + t2[TN:, DH:]) * (1.0 / denom)
        outs.append(oh)

    of = jnp.concatenate(outs, axis=-1)
    out = lax.dot_general(of, wo_ref[...], _DN,
                          preferred_element_type=jnp.float32) + bo_ref[...]
    rowid = lax.broadcasted_iota(jnp.int32, (TN, 1), 0)
    o_ref[0] = jnp.where(rowid < nc, out, 0.0)


def kernel(x_new, rotary_cos, rotary_sin, past_k, past_v,
           Wq, bq, Wk, bk, Wv, bv, Wo, bo,
           past_lengths, new_token_counts, valid_new_mask):
    xf = x_new.reshape(B * TN, D)
    cosf = jnp.tile(rotary_cos.reshape(TN, DH), (B, 1))
    sin1 = rotary_sin.reshape(TN, DH)
    sinf = jnp.tile(jnp.concatenate([-sin1[:, :DH // 2], sin1[:, DH // 2:]],
                                    axis=-1), (B, 1))

    q, kn, vn = pl.pallas_call(
        _proj_kernel,
        out_shape=[jax.ShapeDtypeStruct((B * TN, D), jnp.float32)] * 3,
        name="qkv_rope_proj",
    )(xf, Wq, Wk, Wv, bq.reshape(1, D), bk.reshape(1, D), bv.reshape(1, D),
      cosf, sinf)

    lens = jnp.stack([past_lengths, new_token_counts]).astype(jnp.int32)

    out = pl.pallas_call(
        _attn_kernel,
        grid_spec=pltpu.PrefetchScalarGridSpec(
            num_scalar_prefetch=1,
            grid=(B,),
            in_specs=[
                pl.BlockSpec((1, TN, D), lambda b, lens: (b, 0, 0)),
                pl.BlockSpec((1, H, TN // 2, 2 * DH), lambda b, lens: (b, 0, 0, 0)),
                pl.BlockSpec((1, H, TN // 2, 2 * DH), lambda b, lens: (b, 0, 0, 0)),
                pl.BlockSpec((1, H, LP // 2, 2 * DH), lambda b, lens: (b, 0, 0, 0)),
                pl.BlockSpec((1, H, LP // 2, 2 * DH), lambda b, lens: (b, 0, 0, 0)),
                pl.BlockSpec((D, D), lambda b, lens: (0, 0)),
                pl.BlockSpec((1, D), lambda b, lens: (0, 0)),
            ],
            out_specs=pl.BlockSpec((1, TN, D), lambda b, lens: (b, 0, 0)),
        ),
        out_shape=jax.ShapeDtypeStruct((B, TN, D), jnp.float32),
        compiler_params=pltpu.CompilerParams(
            dimension_semantics=("parallel",),
            vmem_limit_bytes=50 * 1024 * 1024,
        ),
        name="ragged_attn",
    )(lens, q.reshape(B, TN, D),
      kn.reshape(B, TN, H, DH).transpose(0, 2, 1, 3).reshape(B, H, TN // 2, 2 * DH),
      vn.reshape(B, TN, H, DH).transpose(0, 2, 1, 3).reshape(B, H, TN // 2, 2 * DH),
      past_k.reshape(B, H, LP // 2, 2 * DH), past_v.reshape(B, H, LP // 2, 2 * DH),
      Wo, bo.reshape(1, D))

    return out


# speedup vs baseline: 3.1602x; 1.0824x over previous
import jax
import jax.numpy as jnp
from jax import lax
from jax.experimental import pallas as pl
from jax.experimental.pallas import tpu as pltpu

B, TN, D = 16, 16, 1024
H, DH = 16, 64
LP = 2048
NC = 4
CP = LP // NC
NEG = -1e30
_DN = (((1,), (1,)), ((), ()))


def _proj_kernel(x_ref, wq_ref, wk_ref, wv_ref, bq_ref, bk_ref, bv_ref,
                 cos_ref, sin_ref, q_ref, k_ref, v_ref):
    x = x_ref[...]
    cos = cos_ref[...]
    sin = sin_ref[...]

    def rope(y):
        parts = []
        for h in range(H):
            yh = y[:, h * DH:(h + 1) * DH]
            rh = jnp.concatenate([yh[:, DH // 2:], yh[:, :DH // 2]], axis=-1)
            parts.append(yh * cos + rh * sin)
        return jnp.concatenate(parts, axis=-1)

    q = lax.dot_general(x, wq_ref[...], _DN, preferred_element_type=jnp.float32) + bq_ref[...]
    q_ref[...] = rope(q)
    k = lax.dot_general(x, wk_ref[...], _DN, preferred_element_type=jnp.float32) + bk_ref[...]
    k_ref[...] = rope(k)
    v_ref[...] = lax.dot_general(x, wv_ref[...], _DN, preferred_element_type=jnp.float32) + bv_ref[...]


def _attn_kernel(lens_ref, q_ref, kn_ref, vn_ref, pk_ref, pv_ref,
                 wo_ref, bo_ref, o_ref,
                 m_s, l_s, acc_s):
    b = pl.program_id(0)
    i = pl.program_id(1)
    plen = lens_ref[0, b]
    nc = lens_ref[1, b]
    scale = 1.0 / 8.0
    q = q_ref[0]

    @pl.when(i == 0)
    def _init():
        m_s[...] = jnp.full_like(m_s, NEG)
        l_s[...] = jnp.zeros_like(l_s)
        acc_s[...] = jnp.zeros_like(acc_s)

    def online_update(kc_ref, vc_ref, bias, cw):
        for h in range(H):
            sl = slice(h * DH, (h + 1) * DH)
            qh = q[:, sl] * scale
            s = lax.dot_general(qh, kc_ref[h], _DN,
                                preferred_element_type=jnp.float32) + bias
            mc = jnp.max(s, axis=-1, keepdims=True)
            m_old = m_s[h][:, :1]
            m_new = jnp.maximum(m_old, mc)
            alpha = jnp.exp(m_old - m_new)
            p = jnp.exp(s - m_new)
            l_new = (l_s[h][:, :1] * alpha
                     + jnp.sum(p, axis=-1, keepdims=True))
            t = jnp.dot(p, vc_ref[h], preferred_element_type=jnp.float32)
            acc_s[:, sl] = acc_s[:, sl] * alpha + t
            m_s[h] = jnp.broadcast_to(m_new, (TN, 128))
            l_s[h] = jnp.broadcast_to(l_new, (TN, 128))

    @pl.when(CP * i < plen)
    def _past_chunk():
        pos = CP * i + lax.broadcasted_iota(jnp.int32, (1, CP), 1)
        bias = jnp.where(pos < plen, 0.0, NEG)
        online_update(pk_ref.at[0], pv_ref.at[0], bias, CP)

    @pl.when(i == NC - 1)
    def _final():
        tpos = lax.broadcasted_iota(jnp.int32, (1, TN), 1)
        bias = jnp.where(tpos < nc, 0.0, NEG)
        online_update(kn_ref.at[0], vn_ref.at[0], bias, TN)

        linv_parts = []
        for h in range(H):
            linv_parts.append(jnp.broadcast_to(1.0 / l_s[h][:, :1], (TN, DH)))
        of = acc_s[...] * jnp.concatenate(linv_parts, axis=-1)
        out = lax.dot_general(of, wo_ref[...], _DN,
                              preferred_element_type=jnp.float32) + bo_ref[...]
        rowid = lax.broadcasted_iota(jnp.int32, (TN, 1), 0)
        o_ref[0] = jnp.where(rowid < nc, out, 0.0)


def _kv_map(b, i, lens):
    plen = lens[0, b]
    nchunks = (plen + (CP - 1)) // CP
    last = jnp.maximum(nchunks - 1, 0)
    return (b, 0, jnp.minimum(i, last), 0)


def kernel(x_new, rotary_cos, rotary_sin, past_k, past_v,
           Wq, bq, Wk, bk, Wv, bv, Wo, bo,
           past_lengths, new_token_counts, valid_new_mask):
    xf = x_new.reshape(B * TN, D)
    cosf = jnp.tile(rotary_cos.reshape(TN, DH), (B, 1))
    sin1 = rotary_sin.reshape(TN, DH)
    sinf = jnp.tile(jnp.concatenate([-sin1[:, :DH // 2], sin1[:, DH // 2:]],
                                    axis=-1), (B, 1))

    q, kn, vn = pl.pallas_call(
        _proj_kernel,
        out_shape=[jax.ShapeDtypeStruct((B * TN, D), jnp.float32)] * 3,
        name="qkv_rope_proj",
    )(xf, Wq, Wk, Wv, bq.reshape(1, D), bk.reshape(1, D), bv.reshape(1, D),
      cosf, sinf)

    lens = jnp.stack([past_lengths, new_token_counts]).astype(jnp.int32)

    out = pl.pallas_call(
        _attn_kernel,
        grid_spec=pltpu.PrefetchScalarGridSpec(
            num_scalar_prefetch=1,
            grid=(B, NC),
            in_specs=[
                pl.BlockSpec((1, TN, D), lambda b, i, lens: (b, 0, 0)),
                pl.BlockSpec((1, H, TN, DH), lambda b, i, lens: (b, 0, 0, 0)),
                pl.BlockSpec((1, H, TN, DH), lambda b, i, lens: (b, 0, 0, 0)),
                pl.BlockSpec((1, H, CP, DH), _kv_map),
                pl.BlockSpec((1, H, CP, DH), _kv_map),
                pl.BlockSpec((D, D), lambda b, i, lens: (0, 0)),
                pl.BlockSpec((1, D), lambda b, i, lens: (0, 0)),
            ],
            out_specs=pl.BlockSpec((1, TN, D), lambda b, i, lens: (b, 0, 0)),
            scratch_shapes=[
                pltpu.VMEM((H, TN, 128), jnp.float32),
                pltpu.VMEM((H, TN, 128), jnp.float32),
                pltpu.VMEM((TN, D), jnp.float32),
            ],
        ),
        out_shape=jax.ShapeDtypeStruct((B, TN, D), jnp.float32),
        compiler_params=pltpu.CompilerParams(
            dimension_semantics=("parallel", "arbitrary"),
            vmem_limit_bytes=50 * 1024 * 1024,
        ),
        name="ragged_attn",
    )(lens, q.reshape(B, TN, D),
      kn.reshape(B, TN, H, DH).transpose(0, 2, 1, 3),
      vn.reshape(B, TN, H, DH).transpose(0, 2, 1, 3),
      past_k, past_v,
      Wo, bo.reshape(1, D))

    return out


# speedup vs baseline: 19.4173x; 6.1443x over previous
import jax
import jax.numpy as jnp
from jax import lax
from jax.experimental import pallas as pl
from jax.experimental.pallas import tpu as pltpu

B, TN, D = 16, 16, 1024
H, DH = 16, 64
LP = 2048
NEG = -1e30
_DN = (((1,), (1,)), ((), ()))


def _proj_kernel(x_ref, wq_ref, wk_ref, wv_ref, bq_ref, bk_ref, bv_ref,
                 cos_ref, sin_ref, q_ref, k_ref, v_ref):
    x = x_ref[...]
    cos = cos_ref[...]
    sin = sin_ref[...]

    def rope(y):
        parts = []
        for h in range(H):
            yh = y[:, h * DH:(h + 1) * DH]
            rh = jnp.concatenate([yh[:, DH // 2:], yh[:, :DH // 2]], axis=-1)
            parts.append(yh * cos + rh * sin)
        return jnp.concatenate(parts, axis=-1)

    q = lax.dot_general(x, wq_ref[...], _DN, preferred_element_type=jnp.float32) + bq_ref[...]
    q_ref[...] = rope(q)
    k = lax.dot_general(x, wk_ref[...], _DN, preferred_element_type=jnp.float32) + bk_ref[...]
    k_ref[...] = rope(k)
    v_ref[...] = lax.dot_general(x, wv_ref[...], _DN, preferred_element_type=jnp.float32) + bv_ref[...]


def _attn_kernel(lens_ref, q_ref, kn_ref, vn_ref, kt_ref, vt_ref,
                 wo_ref, bo_ref, o_ref):
    b = pl.program_id(0)
    plen = lens_ref[0, b]
    nc = lens_ref[1, b]
    q = q_ref[0]
    kn = kn_ref[0]
    vn = vn_ref[0]

    pcol = lax.broadcasted_iota(jnp.int32, (1, LP), 1)
    ncol = lax.broadcasted_iota(jnp.int32, (1, TN), 1)
    bias = jnp.concatenate([jnp.where(pcol < plen, 0.0, NEG),
                            jnp.where(ncol < nc, 0.0, NEG)], axis=1)
    scale = 1.0 / 8.0

    outs = []
    for h in range(H):
        sl = slice(h * DH, (h + 1) * DH)
        qh = q[:, sl] * scale
        sp = jnp.dot(qh, kt_ref[0, h],
                     preferred_element_type=jnp.float32)
        sn = lax.dot_general(qh, kn[:, sl], _DN,
                             preferred_element_type=jnp.float32)
        s = jnp.concatenate([sp, sn], axis=1) + bias
        m = jnp.max(s, axis=-1, keepdims=True)
        p = jnp.exp(s - m)
        denom = jnp.sum(p, axis=-1, keepdims=True)
        oh = (lax.dot_general(p[:, :LP], vt_ref[0, h], _DN,
                              preferred_element_type=jnp.float32)
              + jnp.dot(p[:, LP:], vn[:, sl],
                        preferred_element_type=jnp.float32)) * (1.0 / denom)
        outs.append(oh)

    of = jnp.concatenate(outs, axis=-1)
    out = lax.dot_general(of, wo_ref[...], _DN,
                          preferred_element_type=jnp.float32) + bo_ref[...]
    rowid = lax.broadcasted_iota(jnp.int32, (TN, 1), 0)
    o_ref[0] = jnp.where(rowid < nc, out, 0.0)


def kernel(x_new, rotary_cos, rotary_sin, past_k, past_v,
           Wq, bq, Wk, bk, Wv, bv, Wo, bo,
           past_lengths, new_token_counts, valid_new_mask):
    xf = x_new.reshape(B * TN, D)
    cosf = jnp.tile(rotary_cos.reshape(TN, DH), (B, 1))
    sin1 = rotary_sin.reshape(TN, DH)
    sinf = jnp.tile(jnp.concatenate([-sin1[:, :DH // 2], sin1[:, DH // 2:]],
                                    axis=-1), (B, 1))

    q, kn, vn = pl.pallas_call(
        _proj_kernel,
        out_shape=[jax.ShapeDtypeStruct((B * TN, D), jnp.float32)] * 3,
        name="qkv_rope_proj",
    )(xf, Wq, Wk, Wv, bq.reshape(1, D), bk.reshape(1, D), bv.reshape(1, D),
      cosf, sinf)

    lens = jnp.stack([past_lengths, new_token_counts]).astype(jnp.int32)

    out = pl.pallas_call(
        _attn_kernel,
        grid_spec=pltpu.PrefetchScalarGridSpec(
            num_scalar_prefetch=1,
            grid=(B,),
            in_specs=[
                pl.BlockSpec((1, TN, D), lambda b, lens: (b, 0, 0)),
                pl.BlockSpec((1, TN, D), lambda b, lens: (b, 0, 0)),
                pl.BlockSpec((1, TN, D), lambda b, lens: (b, 0, 0)),
                pl.BlockSpec((1, H, DH, LP), lambda b, lens: (b, 0, 0, 0)),
                pl.BlockSpec((1, H, DH, LP), lambda b, lens: (b, 0, 0, 0)),
                pl.BlockSpec((D, D), lambda b, lens: (0, 0)),
                pl.BlockSpec((1, D), lambda b, lens: (0, 0)),
            ],
            out_specs=pl.BlockSpec((1, TN, D), lambda b, lens: (b, 0, 0)),
        ),
        out_shape=jax.ShapeDtypeStruct((B, TN, D), jnp.float32),
        compiler_params=pltpu.CompilerParams(
            dimension_semantics=("parallel",),
            vmem_limit_bytes=50 * 1024 * 1024,
        ),
        name="ragged_attn",
    )(lens, q.reshape(B, TN, D), kn.reshape(B, TN, D), vn.reshape(B, TN, D),
      jnp.swapaxes(past_k, 2, 3), jnp.swapaxes(past_v, 2, 3),
      Wo, bo.reshape(1, D))

    return out
